# baseline (speedup 1.0000x reference)
import functools

import jax
import jax.numpy as jnp
from jax import lax
from jax.experimental import pallas as pl
from jax.experimental.pallas import tpu as pltpu
from jax.experimental.pallas import tpu_sc as plsc

_NC = 2
_NS = 16
_NW = _NC * _NS
_CH = 80
_GCH = 80


def _mesh():
    return plsc.VectorSubcoreMesh(core_axis_name="c", subcore_axis_name="s")


@functools.lru_cache(maxsize=None)
def _make_spmm(n, npad, d, e, with_count):
    ec = 125
    epw = e // _NW
    nec = epw // ec
    assert epw * _NW == e and nec * ec == epw and nec % 2 == 0
    wr = npad // _NS
    wch = wr // _CH
    assert wch * _CH == wr and wr % 8 == 0

    out_type = [jax.ShapeDtypeStruct((_NC * npad, d), jnp.float32)]
    scratch = [
        pltpu.VMEM_SHARED((npad, d), jnp.float32),
        pltpu.VMEM((2, ec), jnp.int32),
        pltpu.VMEM((2, ec), jnp.int32),
        pltpu.VMEM((ec, d), jnp.float32),
        pltpu.VMEM((ec, d), jnp.float32),
        pltpu.SemaphoreType.DMA,
        pltpu.SemaphoreType.DMA,
        pltpu.SemaphoreType.DMA,
        pltpu.SemaphoreType.DMA,
        pltpu.SemaphoreType.DMA,
    ]
    if with_count:
        out_type.append(jax.ShapeDtypeStruct((_NC * npad,), jnp.float32))
        scratch.append(pltpu.VMEM((ec,), jnp.float32))
        scratch.append(pltpu.VMEM((wr,), jnp.float32))
        scratch.append(pltpu.VMEM_SHARED((npad,), jnp.float32))

    def body(feat, src3, dst3, zf, out, srcr, dstr, rows, rows2, acc,
             sem, sem2, semw, semi, semi2, zc=None, of=None, cnt_out=None,
             ones=None, crow=None, cacc=None):
        c = lax.axis_index("c")
        s = lax.axis_index("s")
        wid = s * _NC + c

        def i_start(jj, b, sm):
            pltpu.async_copy(src3.at[wid, jj], srcr.at[b], sm)
            pltpu.async_copy(dst3.at[wid, jj], dstr.at[b], sm)

        def i_wait(b, sm):
            pltpu.make_async_copy(src3.at[wid, 0], srcr.at[b], sm).wait()
            pltpu.make_async_copy(dst3.at[wid, 0], dstr.at[b], sm).wait()

        def g_start(b, buf, sm):
            pltpu.async_copy(feat.at[srcr.at[b]], buf, sm)

        def g_wait(buf, sm):
            pltpu.make_async_copy(feat.at[srcr.at[0]], buf, sm).wait()

        def sc_add(buf, b):
            pltpu.sync_copy(buf, acc.at[dstr.at[b]], add=True)
            if with_count:
                pltpu.sync_copy(ones, cacc.at[dstr.at[b]], add=True)

        zrow = rows.at[pl.ds(0, _CH)]
        pltpu.sync_copy(zf, zrow)
        for t in range(wch):
            pltpu.async_copy(zrow, acc.at[pl.ds(s * wr + t * _CH, _CH)],
                             semw)
        if with_count:
            pltpu.sync_copy(zc, crow)
            pltpu.async_copy(crow, cacc.at[pl.ds(s * wr, wr)], semw)
            pltpu.sync_copy(of, ones)
        for t in range(wch):
            pltpu.make_async_copy(
                zrow, acc.at[pl.ds(s * wr, _CH)], semw).wait()
        if with_count:
            pltpu.make_async_copy(
                crow, cacc.at[pl.ds(s * wr, wr)], semw).wait()
        plsc.subcore_barrier()

        i_start(0, 0, semi)
        i_start(1, 1, semi2)
        i_wait(0, semi)
        g_start(0, rows, sem)
        i_wait(1, semi2)
        g_start(1, rows2, sem2)

        def step(i, carry):
            j = 2 * i
            g_wait(rows, sem)
            sc_add(rows, 0)
            i_start(jnp.minimum(j + 2, nec - 1), 0, semi)
            g_wait(rows2, sem2)
            sc_add(rows2, 1)
            i_start(jnp.minimum(j + 3, nec - 1), 1, semi2)
            i_wait(0, semi)
            g_start(0, rows, sem)
            i_wait(1, semi2)
            g_start(1, rows2, sem2)
            return carry

        lax.fori_loop(0, nec // 2, step, 0)
        g_wait(rows, sem)
        g_wait(rows2, sem2)
        plsc.subcore_barrier()

        bufs = (rows.at[pl.ds(0, _CH)], rows2.at[pl.ds(0, _CH)])
        for t in range(wch):
            cur = bufs[t % 2]
            if t >= 2:
                pltpu.make_async_copy(
                    cur, out.at[pl.ds(c * npad, _CH)], semw).wait()
            r0 = s * wr + t * _CH
            pltpu.sync_copy(acc.at[pl.ds(r0, _CH)], cur)
            pltpu.async_copy(cur, out.at[pl.ds(c * npad + r0, _CH)], semw)
        for t in range(min(wch, 2)):
            pltpu.make_async_copy(
                bufs[0], out.at[pl.ds(c * npad, _CH)], semw).wait()
        if with_count:
            pltpu.sync_copy(cacc.at[pl.ds(s * wr, wr)], crow)
            pltpu.sync_copy(crow, cnt_out.at[pl.ds(c * npad + s * wr, wr)])

    if with_count:
        @functools.partial(pl.kernel, out_type=out_type, mesh=_mesh(),
                           scratch_types=scratch)
        def k(feat, src3, dst3, zf, zc, of, out, cnt_out,
              acc, srcr, dstr, rows, rows2, sem, sem2, semw, semi, semi2,
              ones, crow, cacc):
            body(feat, src3, dst3, zf, out, srcr, dstr, rows, rows2, acc,
                 sem, sem2, semw, semi, semi2, zc=zc, of=of,
                 cnt_out=cnt_out, ones=ones, crow=crow, cacc=cacc)
        return k

    @functools.partial(pl.kernel, out_type=out_type, mesh=_mesh(),
                       scratch_types=scratch)
    def k(feat, src3, dst3, zf, out, acc, srcr, dstr, rows, rows2,
          sem, sem2, semw, semi, semi2):
        body(feat, src3, dst3, zf, out, srcr, dstr, rows, rows2, acc,
             sem, sem2, semw, semi, semi2)
    return k


@functools.lru_cache(maxsize=None)
def _make_rowgather(npad, d):
    per = npad // _NW
    nch = per // _GCH
    assert per * _NW == npad and nch * _GCH == per

    @functools.partial(
        pl.kernel,
        out_type=jax.ShapeDtypeStruct((npad, d), jnp.float32),
        mesh=_mesh(),
        scratch_types=[
            pltpu.VMEM((nch, _GCH), jnp.int32),
            pltpu.VMEM((_GCH, d), jnp.float32),
            pltpu.SemaphoreType.DMA,
        ])
    def k(feat, idx3, out, idx_v, rows, sem):
        c = lax.axis_index("c")
        s = lax.axis_index("s")
        wid = s * _NC + c
        pltpu.sync_copy(idx3.at[wid], idx_v)
        for j in range(nch):
            pltpu.async_copy(feat.at[idx_v.at[j]], rows, sem).wait()
            pltpu.sync_copy(rows, out.at[pl.ds(wid * per + j * _GCH, _GCH)])
    return k


@functools.lru_cache(maxsize=None)
def _make_layer(n, d, h, b=400):
    assert n % b == 0

    def body(agg_ref, cnt_ref, x_ref, wl_ref, bl_ref, wr_ref, a_ref,
             h_ref, sum_ref):
        i = pl.program_id(0)
        agg = agg_ref[0] + agg_ref[1]
        cnt = cnt_ref[...]
        denom = jnp.maximum(cnt[:, 0:1] + cnt[:, 1:2], 1.0)
        mean = agg / denom
        out = (jnp.dot(mean, wl_ref[...], preferred_element_type=jnp.float32)
               + bl_ref[...]
               + jnp.dot(x_ref[...], wr_ref[...],
                         preferred_element_type=jnp.float32))
        nrm = jnp.sqrt(jnp.sum(out * out, axis=1, keepdims=True))
        out = out / jnp.maximum(nrm, 1e-12)
        hh = jnp.where(out >= 0, out, a_ref[...] * out)
        h_ref[...] = hh

        @pl.when(i == 0)
        def _():
            sum_ref[...] = jnp.zeros_like(sum_ref)
        sum_ref[...] += jnp.sum(hh, axis=0, keepdims=True)

    return pl.pallas_call(
        body,
        grid=(n // b,),
        in_specs=[
            pl.BlockSpec((2, b, d), lambda i: (0, i, 0)),
            pl.BlockSpec((b, 2), lambda i: (i, 0)),
            pl.BlockSpec((b, d), lambda i: (i, 0)),
            pl.BlockSpec((d, h), lambda i: (0, 0)),
            pl.BlockSpec((1, h), lambda i: (0, 0)),
            pl.BlockSpec((d, h), lambda i: (0, 0)),
            pl.BlockSpec((1, h), lambda i: (0, 0)),
        ],
        out_specs=[pl.BlockSpec((b, h), lambda i: (i, 0)),
                   pl.BlockSpec((1, h), lambda i: (0, 0))],
        out_shape=[jax.ShapeDtypeStruct((n, h), jnp.float32),
                   jax.ShapeDtypeStruct((1, h), jnp.float32)],
    )


@functools.lru_cache(maxsize=None)
def _make_final(n, h, b=400):
    assert n % b == 0
    inv_n = 1.0 / n

    def body(pos_ref, neg_ref, psum_ref, wd_ref, nc_ref, out_ref):
        i = pl.program_id(0)
        summary = 1.0 / (1.0 + jnp.exp(-psum_ref[...] / nc_ref[...]))
        sv = jnp.sum(wd_ref[...] * summary, axis=1)
        ps = jnp.sum(pos_ref[...] * sv[None, :], axis=1)
        ns = jnp.sum(neg_ref[...] * sv[None, :], axis=1)

        def sp(v):
            return jnp.maximum(v, 0.0) + jnp.log(1.0 + jnp.exp(-jnp.abs(v)))

        part = (jnp.sum(sp(-ps)) + jnp.sum(sp(ns))) * inv_n

        @pl.when(i == 0)
        def _():
            out_ref[...] = jnp.zeros_like(out_ref)
        out_ref[...] += part

    return pl.pallas_call(
        body,
        grid=(n // b,),
        in_specs=[
            pl.BlockSpec((b, h), lambda i: (i, 0)),
            pl.BlockSpec((b, h), lambda i: (i, 0)),
            pl.BlockSpec((1, h), lambda i: (0, 0)),
            pl.BlockSpec((h, h), lambda i: (0, 0)),
            pl.BlockSpec((1, 1), lambda i: (0, 0)),
        ],
        out_specs=pl.BlockSpec((1, 1), lambda i: (0, 0)),
        out_shape=jax.ShapeDtypeStruct((1, 1), jnp.float32),
    )


def kernel(x, edge_index, node_cnt, Wl1, bl1, Wr1, a1, Wl2, bl2, Wr2, a2, Wd):
    n, d = x.shape
    h = Wl1.shape[1]
    e = edge_index.shape[1]

    src3 = edge_index[0].reshape(_NW, -1, 125)
    dst3 = edge_index[1].reshape(_NW, -1, 125)

    perm = jax.random.permutation(jax.random.key(42), n)
    unit = _NW * _GCH
    npad = ((n + unit - 1) // unit) * unit
    pad = npad - n
    perm_pad = jnp.concatenate(
        [perm.astype(jnp.int32),
         (jnp.arange(pad, dtype=jnp.int32) % n)]).reshape(_NW, -1, _GCH)

    xp = _make_rowgather(npad, d)(x, perm_pad)

    zf = jnp.zeros((_CH, d), jnp.float32)
    zc = jnp.zeros((npad // _NS,), jnp.float32)
    of = jnp.ones((125,), jnp.float32)

    aggx, cntf = _make_spmm(n, npad, d, e, True)(x, src3, dst3, zf, zc, of)
    (aggxp,) = _make_spmm(n, npad, d, e, False)(xp, src3, dst3, zf)
    cnt2 = jnp.transpose(cntf.reshape(2, npad))

    layer1 = _make_layer(n, d, h)
    bl1r, a1r = bl1.reshape(1, h), a1.reshape(1, h)
    h_pos, _ = layer1(aggx.reshape(2, npad, d), cnt2, x, Wl1, bl1r, Wr1, a1r)
    h_neg, _ = layer1(aggxp.reshape(2, npad, d), cnt2, xp, Wl1, bl1r, Wr1,
                      a1r)

    (aggh,) = _make_spmm(n, npad, h, e, False)(h_pos, src3, dst3, zf)
    (aggg,) = _make_spmm(n, npad, h, e, False)(h_neg, src3, dst3, zf)

    layer2 = _make_layer(n, h, h)
    bl2r, a2r = bl2.reshape(1, h), a2.reshape(1, h)
    pos, psum = layer2(aggh.reshape(2, npad, h), cnt2, h_pos, Wl2, bl2r,
                       Wr2, a2r)
    neg, _ = layer2(aggg.reshape(2, npad, h), cnt2, h_neg, Wl2, bl2r,
                    Wr2, a2r)

    ncf = jnp.asarray(node_cnt, jnp.float32).reshape(1, 1)
    loss = _make_final(n, h)(pos, neg, psum, Wd, ncf)
    return loss.reshape(())


# speedup vs baseline: 1.1391x; 1.1391x over previous
import functools

import jax
import jax.numpy as jnp
from jax import lax
from jax.experimental import pallas as pl
from jax.experimental.pallas import tpu as pltpu
from jax.experimental.pallas import tpu_sc as plsc

_NC = 2
_NS = 16
_NW = _NC * _NS
_CH = 80
_GCH = 80


def _mesh():
    return plsc.VectorSubcoreMesh(core_axis_name="c", subcore_axis_name="s")


@functools.lru_cache(maxsize=None)
def _make_spmm(n, npad, d, e, with_count):
    epw = e // _NW
    nch = epw // _CH
    assert epw * _NW == e and nch * _CH == epw and nch % 2 == 1
    wr = npad // _NS
    wch = wr // _CH
    assert wch * _CH == wr and wr % 8 == 0

    out_type = [jax.ShapeDtypeStruct((_NC * npad, d), jnp.float32)]
    scratch = [
        pltpu.VMEM_SHARED((npad, d), jnp.float32),
        pltpu.VMEM((epw,), jnp.int32),
        pltpu.VMEM((nch, _CH), jnp.int32),
        pltpu.VMEM((_CH, d), jnp.float32),
        pltpu.VMEM((_CH, d), jnp.float32),
        pltpu.SemaphoreType.DMA,
        pltpu.SemaphoreType.DMA,
        pltpu.SemaphoreType.DMA,
    ]
    if with_count:
        out_type.append(jax.ShapeDtypeStruct((_NC * npad,), jnp.float32))
        scratch.append(pltpu.VMEM((_CH,), jnp.float32))
        scratch.append(pltpu.VMEM((wr,), jnp.float32))
        scratch.append(pltpu.VMEM_SHARED((npad,), jnp.float32))

    def body(feat, src2, dst3, zf, out, src_v, dst_v, rows, rows2, acc,
             sem, sem2, semw, zc=None, of=None, cnt_out=None, ones=None,
             crow=None, cacc=None):
        c = lax.axis_index("c")
        s = lax.axis_index("s")
        wid = s * _NC + c

        def g_start(jj, buf, sm):
            pltpu.async_copy(feat.at[src_v.at[pl.ds(jj * _CH, _CH)]],
                             buf, sm)

        def g_wait(buf, sm):
            pltpu.make_async_copy(feat.at[src_v.at[pl.ds(0, _CH)]],
                                  buf, sm).wait()

        pltpu.sync_copy(zf, rows)
        for t in range(wch):
            pltpu.async_copy(rows, acc.at[pl.ds(s * wr + t * _CH, _CH)],
                             semw)
        if with_count:
            pltpu.sync_copy(zc, crow)
            pltpu.async_copy(crow, cacc.at[pl.ds(s * wr, wr)], semw)
        pltpu.sync_copy(src2.at[wid], src_v)
        pltpu.sync_copy(dst3.at[wid], dst_v)
        if with_count:
            pltpu.sync_copy(of, ones)
        for t in range(wch):
            pltpu.make_async_copy(
                rows, acc.at[pl.ds(s * wr, _CH)], semw).wait()
        if with_count:
            pltpu.make_async_copy(
                crow, cacc.at[pl.ds(s * wr, wr)], semw).wait()
        plsc.subcore_barrier()

        g_start(0, rows, sem)
        g_start(1, rows2, sem2)

        def step(i, carry):
            j = 2 * i
            g_wait(rows, sem)
            pltpu.sync_copy(rows, acc.at[dst_v.at[j]], add=True)
            if with_count:
                pltpu.sync_copy(ones, cacc.at[dst_v.at[j]], add=True)
            g_start(j + 2, rows, sem)
            g_wait(rows2, sem2)
            pltpu.sync_copy(rows2, acc.at[dst_v.at[j + 1]], add=True)
            if with_count:
                pltpu.sync_copy(ones, cacc.at[dst_v.at[j + 1]], add=True)
            g_start(jnp.minimum(j + 3, nch - 1), rows2, sem2)
            return carry

        lax.fori_loop(0, nch // 2, step, 0)
        g_wait(rows, sem)
        pltpu.sync_copy(rows, acc.at[dst_v.at[nch - 1]], add=True)
        if with_count:
            pltpu.sync_copy(ones, cacc.at[dst_v.at[nch - 1]], add=True)
        g_wait(rows2, sem2)
        plsc.subcore_barrier()

        bufs = (rows, rows2)
        for t in range(wch):
            cur = bufs[t % 2]
            if t >= 2:
                pltpu.make_async_copy(
                    cur, out.at[pl.ds(c * npad, _CH)], semw).wait()
            r0 = s * wr + t * _CH
            pltpu.sync_copy(acc.at[pl.ds(r0, _CH)], cur)
            pltpu.async_copy(cur, out.at[pl.ds(c * npad + r0, _CH)], semw)
        for t in range(min(wch, 2)):
            pltpu.make_async_copy(
                rows, out.at[pl.ds(c * npad, _CH)], semw).wait()
        if with_count:
            pltpu.sync_copy(cacc.at[pl.ds(s * wr, wr)], crow)
            pltpu.sync_copy(crow, cnt_out.at[pl.ds(c * npad + s * wr, wr)])

    if with_count:
        @functools.partial(pl.kernel, out_type=out_type, mesh=_mesh(),
                           scratch_types=scratch)
        def k(feat, src2, dst3, zf, zc, of, out, cnt_out,
              acc, src_v, dst_v, rows, rows2, sem, sem2, semw,
              ones, crow, cacc):
            body(feat, src2, dst3, zf, out, src_v, dst_v, rows, rows2, acc,
                 sem, sem2, semw, zc=zc, of=of, cnt_out=cnt_out, ones=ones,
                 crow=crow, cacc=cacc)
        return k

    @functools.partial(pl.kernel, out_type=out_type, mesh=_mesh(),
                       scratch_types=scratch)
    def k(feat, src2, dst3, zf, out, acc, src_v, dst_v, rows, rows2,
          sem, sem2, semw):
        body(feat, src2, dst3, zf, out, src_v, dst_v, rows, rows2, acc,
             sem, sem2, semw)
    return k


@functools.lru_cache(maxsize=None)
def _make_rowgather(npad, d):
    per = npad // _NW
    nch = per // _GCH
    assert per * _NW == npad and nch * _GCH == per

    @functools.partial(
        pl.kernel,
        out_type=jax.ShapeDtypeStruct((npad, d), jnp.float32),
        mesh=_mesh(),
        scratch_types=[
            pltpu.VMEM((nch, _GCH), jnp.int32),
            pltpu.VMEM((_GCH, d), jnp.float32),
            pltpu.SemaphoreType.DMA,
        ])
    def k(feat, idx3, out, idx_v, rows, sem):
        c = lax.axis_index("c")
        s = lax.axis_index("s")
        wid = s * _NC + c
        pltpu.sync_copy(idx3.at[wid], idx_v)
        for j in range(nch):
            pltpu.async_copy(feat.at[idx_v.at[j]], rows, sem).wait()
            pltpu.sync_copy(rows, out.at[pl.ds(wid * per + j * _GCH, _GCH)])
    return k


@functools.lru_cache(maxsize=None)
def _make_layer(n, d, h, b=400):
    assert n % b == 0

    def body(agg_ref, cnt_ref, x_ref, wl_ref, bl_ref, wr_ref, a_ref,
             h_ref, sum_ref):
        i = pl.program_id(0)
        agg = agg_ref[0] + agg_ref[1]
        cnt = cnt_ref[...]
        denom = jnp.maximum(cnt[:, 0:1] + cnt[:, 1:2], 1.0)
        mean = agg / denom
        out = (jnp.dot(mean, wl_ref[...], preferred_element_type=jnp.float32)
               + bl_ref[...]
               + jnp.dot(x_ref[...], wr_ref[...],
                         preferred_element_type=jnp.float32))
        nrm = jnp.sqrt(jnp.sum(out * out, axis=1, keepdims=True))
        out = out / jnp.maximum(nrm, 1e-12)
        hh = jnp.where(out >= 0, out, a_ref[...] * out)
        h_ref[...] = hh

        @pl.when(i == 0)
        def _():
            sum_ref[...] = jnp.zeros_like(sum_ref)
        sum_ref[...] += jnp.sum(hh, axis=0, keepdims=True)

    return pl.pallas_call(
        body,
        grid=(n // b,),
        in_specs=[
            pl.BlockSpec((2, b, d), lambda i: (0, i, 0)),
            pl.BlockSpec((b, 2), lambda i: (i, 0)),
            pl.BlockSpec((b, d), lambda i: (i, 0)),
            pl.BlockSpec((d, h), lambda i: (0, 0)),
            pl.BlockSpec((1, h), lambda i: (0, 0)),
            pl.BlockSpec((d, h), lambda i: (0, 0)),
            pl.BlockSpec((1, h), lambda i: (0, 0)),
        ],
        out_specs=[pl.BlockSpec((b, h), lambda i: (i, 0)),
                   pl.BlockSpec((1, h), lambda i: (0, 0))],
        out_shape=[jax.ShapeDtypeStruct((n, h), jnp.float32),
                   jax.ShapeDtypeStruct((1, h), jnp.float32)],
    )


@functools.lru_cache(maxsize=None)
def _make_final(n, h, b=400):
    assert n % b == 0
    inv_n = 1.0 / n

    def body(pos_ref, neg_ref, psum_ref, wd_ref, nc_ref, out_ref):
        i = pl.program_id(0)
        summary = 1.0 / (1.0 + jnp.exp(-psum_ref[...] / nc_ref[...]))
        sv = jnp.sum(wd_ref[...] * summary, axis=1)
        ps = jnp.sum(pos_ref[...] * sv[None, :], axis=1)
        ns = jnp.sum(neg_ref[...] * sv[None, :], axis=1)

        def sp(v):
            return jnp.maximum(v, 0.0) + jnp.log(1.0 + jnp.exp(-jnp.abs(v)))

        part = (jnp.sum(sp(-ps)) + jnp.sum(sp(ns))) * inv_n

        @pl.when(i == 0)
        def _():
            out_ref[...] = jnp.zeros_like(out_ref)
        out_ref[...] += part

    return pl.pallas_call(
        body,
        grid=(n // b,),
        in_specs=[
            pl.BlockSpec((b, h), lambda i: (i, 0)),
            pl.BlockSpec((b, h), lambda i: (i, 0)),
            pl.BlockSpec((1, h), lambda i: (0, 0)),
            pl.BlockSpec((h, h), lambda i: (0, 0)),
            pl.BlockSpec((1, 1), lambda i: (0, 0)),
        ],
        out_specs=pl.BlockSpec((1, 1), lambda i: (0, 0)),
        out_shape=jax.ShapeDtypeStruct((1, 1), jnp.float32),
    )


def kernel(x, edge_index, node_cnt, Wl1, bl1, Wr1, a1, Wl2, bl2, Wr2, a2, Wd):
    n, d = x.shape
    h = Wl1.shape[1]
    e = edge_index.shape[1]

    src2 = edge_index[0].reshape(_NW, -1)
    dst3 = edge_index[1].reshape(_NW, -1, _CH)

    perm = jax.random.permutation(jax.random.key(42), n)
    unit = _NW * _GCH
    npad = ((n + unit - 1) // unit) * unit
    pad = npad - n
    perm_pad = jnp.concatenate(
        [perm.astype(jnp.int32),
         (jnp.arange(pad, dtype=jnp.int32) % n)]).reshape(_NW, -1, _GCH)

    xp = _make_rowgather(npad, d)(x, perm_pad)

    zf = jnp.zeros((_CH, d), jnp.float32)
    zc = jnp.zeros((npad // _NS,), jnp.float32)
    of = jnp.ones((_CH,), jnp.float32)

    aggx, cntf = _make_spmm(n, npad, d, e, True)(x, src2, dst3, zf, zc, of)
    (aggxp,) = _make_spmm(n, npad, d, e, False)(xp, src2, dst3, zf)
    cnt2 = jnp.transpose(cntf.reshape(2, npad))

    layer1 = _make_layer(n, d, h)
    bl1r, a1r = bl1.reshape(1, h), a1.reshape(1, h)
    h_pos, _ = layer1(aggx.reshape(2, npad, d), cnt2, x, Wl1, bl1r, Wr1, a1r)
    h_neg, _ = layer1(aggxp.reshape(2, npad, d), cnt2, xp, Wl1, bl1r, Wr1,
                      a1r)

    (aggh,) = _make_spmm(n, npad, h, e, False)(h_pos, src2, dst3, zf)
    (aggg,) = _make_spmm(n, npad, h, e, False)(h_neg, src2, dst3, zf)

    layer2 = _make_layer(n, h, h)
    bl2r, a2r = bl2.reshape(1, h), a2.reshape(1, h)
    pos, psum = layer2(aggh.reshape(2, npad, h), cnt2, h_pos, Wl2, bl2r,
                       Wr2, a2r)
    neg, _ = layer2(aggg.reshape(2, npad, h), cnt2, h_neg, Wl2, bl2r,
                    Wr2, a2r)

    ncf = jnp.asarray(node_cnt, jnp.float32).reshape(1, 1)
    loss = _make_final(n, h)(pos, neg, psum, Wd, ncf)
    return loss.reshape(())
